# baseline (speedup 1.0000x reference)
import functools

import jax
import jax.numpy as jnp
from jax import lax
from jax.experimental import pallas as pl
from jax.experimental.pallas import tpu as pltpu
from jax.experimental.pallas import tpu_sc as plsc


C = 8
L = 16
NW = 32


def _sc_body(x_hbm, pos_hbm, out_hbm, xbuf, posbuf,
             *, batch, seq_len, d_model):
    s_per_w = seq_len // NW
    n_chunks = s_per_w // C
    n_steps = n_chunks * batch
    cd = C * d_model

    wid = lax.axis_index("s") * 2 + lax.axis_index("c")
    s0 = wid * s_per_w

    def step(t, carry):
        b = lax.rem(t, batch)
        c = lax.div(t, batch)
        off = (b * seq_len + s0 + c * C) * d_model
        poff = (s0 + c * C) * d_model

        pltpu.sync_copy(x_hbm.at[pl.ds(off, cd)], xbuf)

        @pl.when(b == 0)
        def _():
            pltpu.sync_copy(pos_hbm.at[pl.ds(poff, cd)], posbuf)

        @plsc.parallel_loop(0, cd // L, unroll=8)
        def _(j):
            o = j * L
            xbuf[pl.ds(o, L)] = xbuf[pl.ds(o, L)] + posbuf[pl.ds(o, L)]

        pltpu.sync_copy(xbuf, out_hbm.at[pl.ds(off, cd)])
        return carry

    lax.fori_loop(0, n_steps, step, 0)


def kernel(x, pos_table):
    batch, seq_len, d_model = x.shape
    rows = batch * seq_len
    xf = x.reshape(rows * d_model)
    pos = pos_table[:seq_len].reshape(seq_len * d_model)
    cd = C * d_model

    mesh = plsc.VectorSubcoreMesh(core_axis_name="c", subcore_axis_name="s")
    body = functools.partial(_sc_body, batch=batch, seq_len=seq_len,
                             d_model=d_model)
    sc = pl.kernel(
        body,
        out_type=jax.ShapeDtypeStruct((rows * d_model,), x.dtype),
        mesh=mesh,
        scratch_types=[
            pltpu.VMEM((cd,), x.dtype),
            pltpu.VMEM((cd,), x.dtype),
        ],
    )
    out = sc(xf, pos)
    return out.reshape(batch, seq_len, d_model)


# speedup vs baseline: 1.3423x; 1.3423x over previous
import functools

import jax
import jax.numpy as jnp
from jax import lax
from jax.experimental import pallas as pl
from jax.experimental.pallas import tpu as pltpu
from jax.experimental.pallas import tpu_sc as plsc


C = 8
L = 16
NW = 32


def _sc_body(x_hbm, pos_hbm, out_hbm,
             xbuf0, xbuf1, obuf0, obuf1, posbuf,
             rsem0, rsem1, wsem0, wsem1,
             *, batch, seq_len, d_model):
    s_per_w = seq_len // NW
    n_chunks = s_per_w // C
    n_steps = n_chunks * batch
    cd = C * d_model

    wid = lax.axis_index("s") * 2 + lax.axis_index("c")
    s0 = wid * s_per_w

    def x_off(t):
        b = lax.rem(t, batch)
        c = lax.div(t, batch)
        return (b * seq_len + s0 + c * C) * d_model

    def rd(t, buf, sem):
        return pltpu.make_async_copy(
            x_hbm.at[pl.ds(x_off(t), cd)], buf, sem)

    def wr(t, buf, sem):
        return pltpu.make_async_copy(
            buf, out_hbm.at[pl.ds(x_off(t), cd)], sem)

    rd(0, xbuf0, rsem0).start()
    rd(1, xbuf1, rsem1).start()

    def half_step(t, p, xbuf, obuf, rsem, wsem):
        b = lax.rem(t, batch)
        c = lax.div(t, batch)

        @pl.when(b == 0)
        def _():
            pltpu.sync_copy(
                pos_hbm.at[pl.ds((s0 + c * C) * d_model, cd)], posbuf)

        rd(t, xbuf, rsem).wait()

        @pl.when(p > 0)
        def _():
            wr(t - 2, obuf, wsem).wait()

        @plsc.parallel_loop(0, cd // L, unroll=8)
        def _(j):
            o = j * L
            obuf[pl.ds(o, L)] = xbuf[pl.ds(o, L)] + posbuf[pl.ds(o, L)]

        wr(t, obuf, wsem).start()

        @pl.when(t + 2 < n_steps)
        def _():
            rd(t + 2, xbuf, rsem).start()

    def step(p, carry):
        half_step(2 * p, p, xbuf0, obuf0, rsem0, wsem0)
        half_step(2 * p + 1, p, xbuf1, obuf1, rsem1, wsem1)
        return carry

    lax.fori_loop(0, n_steps // 2, step, 0)

    wr(n_steps - 2, obuf0, wsem0).wait()
    wr(n_steps - 1, obuf1, wsem1).wait()


def kernel(x, pos_table):
    batch, seq_len, d_model = x.shape
    rows = batch * seq_len
    xf = x.reshape(rows * d_model)
    pos = pos_table[:seq_len].reshape(seq_len * d_model)
    cd = C * d_model

    mesh = plsc.VectorSubcoreMesh(core_axis_name="c", subcore_axis_name="s")
    body = functools.partial(_sc_body, batch=batch, seq_len=seq_len,
                             d_model=d_model)
    sc = pl.kernel(
        body,
        out_type=jax.ShapeDtypeStruct((rows * d_model,), x.dtype),
        mesh=mesh,
        scratch_types=[
            pltpu.VMEM((cd,), x.dtype),
            pltpu.VMEM((cd,), x.dtype),
            pltpu.VMEM((cd,), x.dtype),
            pltpu.VMEM((cd,), x.dtype),
            pltpu.VMEM((cd,), x.dtype),
            pltpu.SemaphoreType.DMA,
            pltpu.SemaphoreType.DMA,
            pltpu.SemaphoreType.DMA,
            pltpu.SemaphoreType.DMA,
        ],
    )
    out = sc(xf, pos)
    return out.reshape(batch, seq_len, d_model)


# speedup vs baseline: 1.4164x; 1.0552x over previous
import functools

import jax
import jax.numpy as jnp
from jax import lax
from jax.experimental import pallas as pl
from jax.experimental.pallas import tpu as pltpu
from jax.experimental.pallas import tpu_sc as plsc


C = 16
L = 16
NW = 32


def _sc_body(x_hbm, pos_hbm, out_hbm,
             xbuf0, xbuf1, obuf0, obuf1, posbuf,
             rsem0, rsem1, wsem0, wsem1,
             *, batch, seq_len, d_model):
    s_per_w = seq_len // NW
    n_chunks = s_per_w // C
    n_steps = n_chunks * batch
    cd = C * d_model

    wid = lax.axis_index("s") * 2 + lax.axis_index("c")
    s0 = wid * s_per_w

    def x_off(t):
        b = lax.rem(t, batch)
        c = lax.div(t, batch)
        return (b * seq_len + s0 + c * C) * d_model

    def rd(t, buf, sem):
        return pltpu.make_async_copy(
            x_hbm.at[pl.ds(x_off(t), cd)], buf, sem)

    def wr(t, buf, sem):
        return pltpu.make_async_copy(
            buf, out_hbm.at[pl.ds(x_off(t), cd)], sem)

    rd(0, xbuf0, rsem0).start()
    rd(1, xbuf1, rsem1).start()

    def half_step(t, p, xbuf, obuf, rsem, wsem):
        b = lax.rem(t, batch)
        c = lax.div(t, batch)

        @pl.when(b == 0)
        def _():
            pltpu.sync_copy(
                pos_hbm.at[pl.ds((s0 + c * C) * d_model, cd)], posbuf)

        rd(t, xbuf, rsem).wait()

        @pl.when(p > 0)
        def _():
            wr(t - 2, obuf, wsem).wait()

        @plsc.parallel_loop(0, cd // L, unroll=8)
        def _(j):
            o = j * L
            obuf[pl.ds(o, L)] = xbuf[pl.ds(o, L)] + posbuf[pl.ds(o, L)]

        wr(t, obuf, wsem).start()

        @pl.when(t + 2 < n_steps)
        def _():
            rd(t + 2, xbuf, rsem).start()

    def step(p, carry):
        half_step(2 * p, p, xbuf0, obuf0, rsem0, wsem0)
        half_step(2 * p + 1, p, xbuf1, obuf1, rsem1, wsem1)
        return carry

    lax.fori_loop(0, n_steps // 2, step, 0)

    wr(n_steps - 2, obuf0, wsem0).wait()
    wr(n_steps - 1, obuf1, wsem1).wait()


def kernel(x, pos_table):
    batch, seq_len, d_model = x.shape
    rows = batch * seq_len
    xf = x.reshape(rows * d_model)
    pos = pos_table[:seq_len].reshape(seq_len * d_model)
    cd = C * d_model

    mesh = plsc.VectorSubcoreMesh(core_axis_name="c", subcore_axis_name="s")
    body = functools.partial(_sc_body, batch=batch, seq_len=seq_len,
                             d_model=d_model)
    sc = pl.kernel(
        body,
        out_type=jax.ShapeDtypeStruct((rows * d_model,), x.dtype),
        mesh=mesh,
        scratch_types=[
            pltpu.VMEM((cd,), x.dtype),
            pltpu.VMEM((cd,), x.dtype),
            pltpu.VMEM((cd,), x.dtype),
            pltpu.VMEM((cd,), x.dtype),
            pltpu.VMEM((cd,), x.dtype),
            pltpu.SemaphoreType.DMA,
            pltpu.SemaphoreType.DMA,
            pltpu.SemaphoreType.DMA,
            pltpu.SemaphoreType.DMA,
        ],
    )
    out = sc(xf, pos)
    return out.reshape(batch, seq_len, d_model)


# speedup vs baseline: 6.1379x; 4.3335x over previous
import functools

import jax
import jax.numpy as jnp
from jax.experimental import pallas as pl
from jax.experimental.pallas import tpu as pltpu


R = 512
K = 4


def _pos_add_body(x_hbm, pos_hbm, out_hbm, posbuf, xbuf, obuf,
                  pos_sems, rd_sems, wr_sems, *, n_chunks, pos_chunks):
    def pos_copy(c):
        return pltpu.make_async_copy(
            pos_hbm.at[pl.ds(c * R, R)], posbuf.at[pl.ds(c * R, R)],
            pos_sems.at[c])

    def rd_copy(i, slot):
        return pltpu.make_async_copy(
            x_hbm.at[pl.ds(i * R, R)], xbuf.at[slot], rd_sems.at[slot])

    def wr_copy(i, slot):
        return pltpu.make_async_copy(
            obuf.at[slot], out_hbm.at[pl.ds(i * R, R)], wr_sems.at[slot])

    for c in range(pos_chunks):
        pos_copy(c).start()
    for i in range(K):
        rd_copy(i, i).start()

    def step(i, _):
        slot = jax.lax.rem(i, K)
        pc = jax.lax.rem(i, pos_chunks)

        @pl.when(i < pos_chunks)
        def _():
            pos_copy(pc).wait()

        rd_copy(i, slot).wait()

        @pl.when(i >= K)
        def _():
            wr_copy(i - K, slot).wait()

        obuf[slot] = xbuf[slot] + posbuf[pl.ds(pc * R, R), :]
        wr_copy(i, slot).start()

        @pl.when(i + K < n_chunks)
        def _():
            rd_copy(i + K, slot).start()

        return 0

    jax.lax.fori_loop(0, n_chunks, step, 0)

    for j in range(K):
        i = n_chunks - K + j
        wr_copy(i, i % K).wait()


def kernel(x, pos_table):
    batch, seq_len, d_model = x.shape
    rows = batch * seq_len
    n_chunks = rows // R
    pos_chunks = seq_len // R
    xf = x.reshape(rows, d_model)
    pos = pos_table[:seq_len]

    body = functools.partial(_pos_add_body, n_chunks=n_chunks,
                             pos_chunks=pos_chunks)
    out = pl.pallas_call(
        body,
        in_specs=[
            pl.BlockSpec(memory_space=pl.ANY),
            pl.BlockSpec(memory_space=pl.ANY),
        ],
        out_specs=pl.BlockSpec(memory_space=pl.ANY),
        out_shape=jax.ShapeDtypeStruct((rows, d_model), x.dtype),
        scratch_shapes=[
            pltpu.VMEM((seq_len, d_model), x.dtype),
            pltpu.VMEM((K, R, d_model), x.dtype),
            pltpu.VMEM((K, R, d_model), x.dtype),
            pltpu.SemaphoreType.DMA((pos_chunks,)),
            pltpu.SemaphoreType.DMA((K,)),
            pltpu.SemaphoreType.DMA((K,)),
        ],
    )(xf, pos)
    return out.reshape(batch, seq_len, d_model)
